# baseline (speedup 1.0000x reference)
import jax
import jax.numpy as jnp
from jax.experimental import pallas as pl
from jax.experimental.pallas import tpu as pltpu

ALPHA = 1.702
LIMIT = 7.0
FFB = 256


def _moe_kernel(x_ref, gw_ref, gb_ref, wg_ref, bg_ref, wu_ref, bu_ref,
                wd_ref, bd_ref, out_ref, wrout_ref, xbf_ref):
    e = pl.program_id(0)
    f = pl.program_id(1)
    T, E = wrout_ref.shape

    @pl.when((e == 0) & (f == 0))
    def _router():
        x = x_ref[...]
        xbf_ref[...] = x.astype(jnp.bfloat16)
        logits = jax.lax.dot_general(
            x.astype(jnp.bfloat16), gw_ref[...].astype(jnp.bfloat16),
            (((1,), (1,)), ((), ())),
            preferred_element_type=jnp.float32) + gb_ref[...]
        lane = jax.lax.broadcasted_iota(jnp.int32, (T, E), 1)
        m1 = jnp.max(logits, axis=1, keepdims=True)
        a1 = jnp.min(jnp.where(logits == m1, lane, E), axis=1, keepdims=True)
        masked = jnp.where(lane == a1, -jnp.inf, logits)
        m2 = jnp.max(masked, axis=1, keepdims=True)
        a2 = jnp.min(jnp.where(masked == m2, lane, E), axis=1, keepdims=True)
        e2 = jnp.exp(m2 - m1)
        denom = 1.0 + e2
        w1 = 1.0 / denom
        w2 = e2 / denom
        wrout_ref[...] = (w1 * (lane == a1) + w2 * (lane == a2)).astype(jnp.float32)

    lane = jax.lax.broadcasted_iota(jnp.int32, (T, E), 1)
    we = jnp.sum(wrout_ref[...] * (lane == e), axis=1, keepdims=True)

    xbf = xbf_ref[...]
    g = jnp.dot(xbf, wg_ref[0].astype(jnp.bfloat16),
                preferred_element_type=jnp.float32) + bg_ref[0, 0]
    u = jnp.dot(xbf, wu_ref[0].astype(jnp.bfloat16),
                preferred_element_type=jnp.float32) + bu_ref[0, 0]
    g = jnp.minimum(g, LIMIT)
    u = jnp.clip(u, -LIMIT, LIMIT)
    glu = g * jax.nn.sigmoid(ALPHA * g)
    gated = (u + 1.0) * glu * we
    partial = jnp.dot(gated.astype(jnp.bfloat16), wd_ref[0].astype(jnp.bfloat16),
                      preferred_element_type=jnp.float32)

    @pl.when((e == 0) & (f == 0))
    def _init():
        out_ref[...] = partial + we * bd_ref[0, 0]

    @pl.when((e > 0) & (f == 0))
    def _bias():
        out_ref[...] += partial + we * bd_ref[0, 0]

    @pl.when(f > 0)
    def _acc():
        out_ref[...] += partial


@jax.jit
def kernel(hidden_states, gate_w, gate_b, Wg, bg, Wu, bu, Wd, bd):
    T, H = hidden_states.shape
    E, _, FF = Wg.shape
    nf = FF // FFB
    return pl.pallas_call(
        _moe_kernel,
        grid=(E, nf),
        in_specs=[
            pl.BlockSpec((T, H), lambda e, f: (0, 0)),
            pl.BlockSpec((E, H), lambda e, f: (0, 0)),
            pl.BlockSpec((1, E), lambda e, f: (0, 0)),
            pl.BlockSpec((1, H, FFB), lambda e, f: (e, 0, f)),
            pl.BlockSpec((1, 1, FFB), lambda e, f: (e, 0, f)),
            pl.BlockSpec((1, H, FFB), lambda e, f: (e, 0, f)),
            pl.BlockSpec((1, 1, FFB), lambda e, f: (e, 0, f)),
            pl.BlockSpec((1, FFB, H), lambda e, f: (e, f, 0)),
            pl.BlockSpec((1, 1, H), lambda e, f: (e, 0, 0)),
        ],
        out_specs=pl.BlockSpec((T, H), lambda e, f: (0, 0)),
        out_shape=jax.ShapeDtypeStruct((T, H), jnp.float32),
        scratch_shapes=[
            pltpu.VMEM((T, E), jnp.float32),
            pltpu.VMEM((T, H), jnp.bfloat16),
        ],
    )(hidden_states, gate_w, gate_b.reshape(1, E), Wg, bg.reshape(E, 1, FF),
      Wu, bu.reshape(E, 1, FF), Wd, bd.reshape(E, 1, H))


# speedup vs baseline: 1.0466x; 1.0466x over previous
import jax
import jax.numpy as jnp
from jax.experimental import pallas as pl
from jax.experimental.pallas import tpu as pltpu
from jax.experimental.pallas import tpu_sc as plsc

ALPHA = 1.702
LIMIT = 7.0
FFB = 512
RB = 128


def _router_kernel(x_ref, gw_ref, gb_ref, wr_ref, sel_ref):
    x = x_ref[...]
    T = x.shape[0]
    E = gw_ref.shape[0]
    logits = jax.lax.dot_general(
        x.astype(jnp.bfloat16), gw_ref[...].astype(jnp.bfloat16),
        (((1,), (1,)), ((), ())),
        preferred_element_type=jnp.float32) + gb_ref[...]
    lane = jax.lax.broadcasted_iota(jnp.int32, (T, E), 1)
    m1 = jnp.max(logits, axis=1, keepdims=True)
    a1 = jnp.min(jnp.where(logits == m1, lane, E), axis=1, keepdims=True)
    masked = jnp.where(lane == a1, -jnp.inf, logits)
    m2 = jnp.max(masked, axis=1, keepdims=True)
    a2 = jnp.min(jnp.where(masked == m2, lane, E), axis=1, keepdims=True)
    e2 = jnp.exp(m2 - m1)
    denom = 1.0 + e2
    w1 = 1.0 / denom
    w2 = e2 / denom
    sel = (lane == a1) | (lane == a2)
    wr_ref[...] = (w1 * (lane == a1) + w2 * (lane == a2)).astype(jnp.float32)
    sel_ref[...] = sel.astype(jnp.int32)


def _router(x, gate_w, gate_b):
    T = x.shape[0]
    E = gate_w.shape[0]
    return pl.pallas_call(
        _router_kernel,
        out_shape=(jax.ShapeDtypeStruct((T, E), jnp.float32),
                   jax.ShapeDtypeStruct((T, E), jnp.int32)),
    )(x, gate_w, gate_b.reshape(1, E))


def _dispatch_sc(wrout, sel):
    T, E = wrout.shape
    epc = E // 2
    mesh = plsc.ScalarSubcoreMesh(axis_name="core", num_cores=2)

    @pl.kernel(
        out_type=(jax.ShapeDtypeStruct((256,), jnp.int32),
                  jax.ShapeDtypeStruct((E * T,), jnp.int32),
                  jax.ShapeDtypeStruct((E * T,), jnp.float32)),
        mesh=mesh,
        scratch_types=[pltpu.SMEM((T * E,), jnp.float32),
                       pltpu.SMEM((T * E,), jnp.int32),
                       pltpu.SMEM((epc * T,), jnp.int32),
                       pltpu.SMEM((epc * T,), jnp.float32),
                       pltpu.SMEM((128,), jnp.int32),
                       pltpu.SMEM((1,), jnp.int32),
                       pltpu.SemaphoreType.DMA],
    )
    def dispatch(wr_hbm, sel_hbm, cnt_hbm, perm_hbm, ws_hbm,
                 wr_s, sel_s, perm_s, ws_s, cnt_s, pos_s, sem):
        core = jax.lax.axis_index("core")
        pltpu.async_copy(wr_hbm, wr_s, sem).wait()
        pltpu.async_copy(sel_hbm, sel_s, sem).wait()

        @pl.loop(0, 128)
        def _zcnt(i):
            cnt_s[i] = 0

        @pl.loop(0, epc)
        def _expert(j):
            e = core * epc + j

            @pl.loop(0, T)
            def _zero(t):
                perm_s[j * T + t] = 0
                ws_s[j * T + t] = 0.0

            pos_s[0] = 0

            @pl.loop(0, T)
            def _scan(t):
                @pl.when(sel_s[t * E + e] == 1)
                def _take():
                    p = pos_s[0]
                    perm_s[j * T + p] = t
                    ws_s[j * T + p] = wr_s[t * E + e]
                    pos_s[0] = p + 1

            cnt_s[j] = pos_s[0]

        blk = pl.ds(core * (epc * T), epc * T)
        pltpu.async_copy(perm_s, perm_hbm.at[blk], sem).wait()
        pltpu.async_copy(ws_s, ws_hbm.at[blk], sem).wait()
        pltpu.async_copy(cnt_s, cnt_hbm.at[pl.ds(core * 128, 128)], sem).wait()

    cnt256, perm, ws = dispatch(wrout.reshape(T * E), sel.reshape(T * E))
    return (cnt256.reshape(2, 128)[:, :epc].reshape(E),
            perm.reshape(E, T), ws.reshape(E, T))


def _moe_kernel(cnt_ref, x_ref, perm_ref, ws_ref, wg_ref, bg_ref, wu_ref,
                bu_ref, wd_ref, bd_ref, out_ref, xbf_ref, xg_ref, yacc_ref):
    e = pl.program_id(0)
    f = pl.program_id(1)
    nf = pl.num_programs(1)
    T, H = x_ref.shape
    R = T // RB
    cnt = cnt_ref[e]

    @pl.when((e == 0) & (f == 0))
    def _first():
        xbf_ref[...] = x_ref[...].astype(jnp.bfloat16)
        out_ref[...] = jnp.zeros_like(out_ref)

    for r in range(R):
        @pl.when(cnt > r * RB)
        def _block(r=r):
            rows = pl.ds(r * RB, RB)
            perm_row = perm_ref[0, r, :][None, :]
            iota_t = jax.lax.broadcasted_iota(jnp.int32, (T, RB), 0)
            oh = (iota_t == perm_row).astype(jnp.float32)

            @pl.when(f == 0)
            def _gather():
                xg = jax.lax.dot_general(
                    oh.astype(jnp.bfloat16), xbf_ref[...],
                    (((0,), (0,)), ((), ())),
                    preferred_element_type=jnp.float32)
                xg_ref[rows, :] = xg.astype(jnp.bfloat16)

            xr = xg_ref[rows, :]
            g = jnp.dot(xr, wg_ref[0].astype(jnp.bfloat16),
                        preferred_element_type=jnp.float32) + bg_ref[0, 0]
            u = jnp.dot(xr, wu_ref[0].astype(jnp.bfloat16),
                        preferred_element_type=jnp.float32) + bu_ref[0, 0]
            g = jnp.minimum(g, LIMIT)
            u = jnp.clip(u, -LIMIT, LIMIT)
            glu = g * jax.nn.sigmoid(ALPHA * g)
            gated = (u + 1.0) * glu
            partial = jnp.dot(gated.astype(jnp.bfloat16),
                              wd_ref[0].astype(jnp.bfloat16),
                              preferred_element_type=jnp.float32)

            @pl.when(f == 0)
            def _y0():
                yacc_ref[rows, :] = partial

            @pl.when(f > 0)
            def _yacc():
                yacc_ref[rows, :] += partial

            @pl.when(f == nf - 1)
            def _combine():
                ohw = oh * ws_ref[0, r, :][None, :]
                yout = yacc_ref[rows, :] + bd_ref[0, 0]
                out_ref[...] += jnp.dot(
                    ohw.astype(jnp.bfloat16), yout.astype(jnp.bfloat16),
                    preferred_element_type=jnp.float32)


@jax.jit
def kernel(hidden_states, gate_w, gate_b, Wg, bg, Wu, bu, Wd, bd):
    T, H = hidden_states.shape
    E, _, FF = Wg.shape
    nf = FF // FFB
    R = T // RB
    wrout, sel = _router(hidden_states, gate_w, gate_b)
    cnt, perm, wsort = _dispatch_sc(wrout, sel)
    grid_spec = pltpu.PrefetchScalarGridSpec(
        num_scalar_prefetch=1,
        grid=(E, nf),
        in_specs=[
            pl.BlockSpec((T, H), lambda e, f, *_: (0, 0)),
            pl.BlockSpec((1, R, RB), lambda e, f, *_: (e, 0, 0)),
            pl.BlockSpec((1, R, RB), lambda e, f, *_: (e, 0, 0)),
            pl.BlockSpec((1, H, FFB), lambda e, f, *_: (e, 0, f)),
            pl.BlockSpec((1, 1, FFB), lambda e, f, *_: (e, 0, f)),
            pl.BlockSpec((1, H, FFB), lambda e, f, *_: (e, 0, f)),
            pl.BlockSpec((1, 1, FFB), lambda e, f, *_: (e, 0, f)),
            pl.BlockSpec((1, FFB, H), lambda e, f, *_: (e, f, 0)),
            pl.BlockSpec((1, 1, H), lambda e, f, *_: (e, 0, 0)),
        ],
        out_specs=pl.BlockSpec((T, H), lambda e, f, *_: (0, 0)),
        scratch_shapes=[
            pltpu.VMEM((T, H), jnp.bfloat16),
            pltpu.VMEM((T, H), jnp.bfloat16),
            pltpu.VMEM((T, H), jnp.float32),
        ],
    )
    return pl.pallas_call(
        _moe_kernel,
        grid_spec=grid_spec,
        out_shape=jax.ShapeDtypeStruct((T, H), jnp.float32),
    )(cnt, hidden_states, perm.reshape(E, R, RB), wsort.reshape(E, R, RB),
      Wg, bg.reshape(E, 1, FF), Wu, bu.reshape(E, 1, FF), Wd,
      bd.reshape(E, 1, H))


# speedup vs baseline: 1.0679x; 1.0203x over previous
import jax
import jax.numpy as jnp
from jax.experimental import pallas as pl
from jax.experimental.pallas import tpu as pltpu
from jax.experimental.pallas import tpu_sc as plsc

ALPHA = 1.702
LIMIT = 7.0
FFB = 512
RB = 128


def _router_kernel(x_ref, gw_ref, gb_ref, pck_ref, w1_ref, w2_ref):
    x = x_ref[...]
    T = x.shape[0]
    E = gw_ref.shape[0]
    logits = jax.lax.dot_general(
        x.astype(jnp.bfloat16), gw_ref[...].astype(jnp.bfloat16),
        (((1,), (1,)), ((), ())),
        preferred_element_type=jnp.float32) + gb_ref[...]
    lane = jax.lax.broadcasted_iota(jnp.int32, (T, E), 1)
    m1 = jnp.max(logits, axis=1, keepdims=True)
    a1 = jnp.min(jnp.where(logits == m1, lane, E), axis=1, keepdims=True)
    masked = jnp.where(lane == a1, -jnp.inf, logits)
    m2 = jnp.max(masked, axis=1, keepdims=True)
    a2 = jnp.min(jnp.where(masked == m2, lane, E), axis=1, keepdims=True)
    e2 = jnp.exp(m2 - m1)
    denom = 1.0 + e2
    w1 = 1.0 / denom
    w2 = e2 / denom
    pck_ref[...] = a1 * E + a2
    w1_ref[...] = w1.astype(jnp.float32)
    w2_ref[...] = w2.astype(jnp.float32)


def _router(x, gate_w, gate_b):
    T = x.shape[0]
    E = gate_w.shape[0]
    return pl.pallas_call(
        _router_kernel,
        out_shape=(jax.ShapeDtypeStruct((T, 1), jnp.int32),
                   jax.ShapeDtypeStruct((T, 1), jnp.float32),
                   jax.ShapeDtypeStruct((T, 1), jnp.float32)),
    )(x, gate_w, gate_b.reshape(1, E))


def _dispatch_sc(pck, w1, w2, E):
    T = pck.shape[0]
    epc = E // 2
    mesh = plsc.ScalarSubcoreMesh(axis_name="core", num_cores=2)

    @pl.kernel(
        out_type=(jax.ShapeDtypeStruct((256,), jnp.int32),
                  jax.ShapeDtypeStruct((E * T,), jnp.int32),
                  jax.ShapeDtypeStruct((E * T,), jnp.float32)),
        mesh=mesh,
        scratch_types=[pltpu.SMEM((T,), jnp.int32),
                       pltpu.SMEM((T,), jnp.float32),
                       pltpu.SMEM((T,), jnp.float32),
                       pltpu.SMEM((epc * T,), jnp.int32),
                       pltpu.SMEM((epc * T,), jnp.float32),
                       pltpu.SMEM((128,), jnp.int32),
                       pltpu.SMEM((epc,), jnp.int32),
                       pltpu.SemaphoreType.DMA],
    )
    def dispatch(pck_hbm, w1_hbm, w2_hbm, cnt_hbm, perm_hbm, ws_hbm,
                 pck_s, w1_s, w2_s, perm_s, ws_s, cnt_s, pos_s, sem):
        core = jax.lax.axis_index("core")
        lo = core * epc
        h1 = pltpu.async_copy(pck_hbm, pck_s, sem)
        h2 = pltpu.async_copy(w1_hbm, w1_s, sem)
        h3 = pltpu.async_copy(w2_hbm, w2_s, sem)
        h1.wait()
        h2.wait()
        h3.wait()

        @pl.loop(0, epc)
        def _zpos(j):
            pos_s[j] = 0

        @pl.loop(0, T)
        def _scan(t):
            p = pck_s[t]
            e1 = p // E
            e2 = p - e1 * E
            l1 = e1 - lo
            l2 = e2 - lo

            @pl.when((l1 >= 0) & (l1 < epc))
            def _take1():
                q = pos_s[l1]
                perm_s[l1 * T + q] = t
                ws_s[l1 * T + q] = w1_s[t]
                pos_s[l1] = q + 1

            @pl.when((l2 >= 0) & (l2 < epc))
            def _take2():
                q = pos_s[l2]
                perm_s[l2 * T + q] = t
                ws_s[l2 * T + q] = w2_s[t]
                pos_s[l2] = q + 1

        @pl.loop(0, epc)
        def _finish(j):
            c = pos_s[j]
            cnt_s[j] = c
            tail = ((c + RB - 1) // RB) * RB

            @pl.loop(0, RB)
            def _ztail(i):
                @pl.when(c + i < tail)
                def _z():
                    ws_s[j * T + c + i] = 0.0

        blk = pl.ds(core * (epc * T), epc * T)
        o1 = pltpu.async_copy(perm_s, perm_hbm.at[blk], sem)
        o2 = pltpu.async_copy(ws_s, ws_hbm.at[blk], sem)
        o3 = pltpu.async_copy(cnt_s, cnt_hbm.at[pl.ds(core * 128, 128)], sem)
        o1.wait()
        o2.wait()
        o3.wait()

    cnt256, perm, ws = dispatch(pck.reshape(T), w1.reshape(T), w2.reshape(T))
    return (cnt256.reshape(2, 128)[:, :epc].reshape(E),
            perm.reshape(E, T), ws.reshape(E, T))


def _moe_kernel(cnt_ref, x_ref, perm_ref, ws_ref, wg_ref, bg_ref, wu_ref,
                bu_ref, wd_ref, bd_ref, out_ref, xbf_ref, xg_ref, yacc_ref):
    e = pl.program_id(0)
    f = pl.program_id(1)
    nf = pl.num_programs(1)
    T, H = x_ref.shape
    R = T // RB
    cnt = cnt_ref[e]

    @pl.when((e == 0) & (f == 0))
    def _first():
        xbf_ref[...] = x_ref[...].astype(jnp.bfloat16)
        out_ref[...] = jnp.zeros_like(out_ref)

    for r in range(R):
        @pl.when(cnt > r * RB)
        def _block(r=r):
            rows = pl.ds(r * RB, RB)
            perm_row = perm_ref[0, r, :][None, :]
            iota_t = jax.lax.broadcasted_iota(jnp.int32, (T, RB), 0)
            oh = (iota_t == perm_row).astype(jnp.float32)

            @pl.when(f == 0)
            def _gather():
                xg = jax.lax.dot_general(
                    oh.astype(jnp.bfloat16), xbf_ref[...],
                    (((0,), (0,)), ((), ())),
                    preferred_element_type=jnp.float32)
                xg_ref[rows, :] = xg.astype(jnp.bfloat16)

            xr = xg_ref[rows, :]
            g = jnp.dot(xr, wg_ref[0].astype(jnp.bfloat16),
                        preferred_element_type=jnp.float32) + bg_ref[0, 0]
            u = jnp.dot(xr, wu_ref[0].astype(jnp.bfloat16),
                        preferred_element_type=jnp.float32) + bu_ref[0, 0]
            g = jnp.minimum(g, LIMIT)
            u = jnp.clip(u, -LIMIT, LIMIT)
            glu = g * jax.nn.sigmoid(ALPHA * g)
            gated = (u + 1.0) * glu
            partial = jnp.dot(gated.astype(jnp.bfloat16),
                              wd_ref[0].astype(jnp.bfloat16),
                              preferred_element_type=jnp.float32)

            @pl.when(f == 0)
            def _y0():
                yacc_ref[rows, :] = partial

            @pl.when(f > 0)
            def _yacc():
                yacc_ref[rows, :] += partial

            @pl.when(f == nf - 1)
            def _combine():
                ohw = oh * ws_ref[0, r, :][None, :]
                yout = yacc_ref[rows, :] + bd_ref[0, 0]
                out_ref[...] += jnp.dot(
                    ohw.astype(jnp.bfloat16), yout.astype(jnp.bfloat16),
                    preferred_element_type=jnp.float32)


@jax.jit
def kernel(hidden_states, gate_w, gate_b, Wg, bg, Wu, bu, Wd, bd):
    T, H = hidden_states.shape
    E, _, FF = Wg.shape
    nf = FF // FFB
    R = T // RB
    pck, w1, w2 = _router(hidden_states, gate_w, gate_b)
    cnt, perm, wsort = _dispatch_sc(pck, w1, w2, E)
    grid_spec = pltpu.PrefetchScalarGridSpec(
        num_scalar_prefetch=1,
        grid=(E, nf),
        in_specs=[
            pl.BlockSpec((T, H), lambda e, f, *_: (0, 0)),
            pl.BlockSpec((1, R, RB), lambda e, f, *_: (e, 0, 0)),
            pl.BlockSpec((1, R, RB), lambda e, f, *_: (e, 0, 0)),
            pl.BlockSpec((1, H, FFB), lambda e, f, *_: (e, 0, f)),
            pl.BlockSpec((1, 1, FFB), lambda e, f, *_: (e, 0, f)),
            pl.BlockSpec((1, H, FFB), lambda e, f, *_: (e, 0, f)),
            pl.BlockSpec((1, 1, FFB), lambda e, f, *_: (e, 0, f)),
            pl.BlockSpec((1, FFB, H), lambda e, f, *_: (e, f, 0)),
            pl.BlockSpec((1, 1, H), lambda e, f, *_: (e, 0, 0)),
        ],
        out_specs=pl.BlockSpec((T, H), lambda e, f, *_: (0, 0)),
        scratch_shapes=[
            pltpu.VMEM((T, H), jnp.bfloat16),
            pltpu.VMEM((T, H), jnp.bfloat16),
            pltpu.VMEM((T, H), jnp.float32),
        ],
    )
    return pl.pallas_call(
        _moe_kernel,
        grid_spec=grid_spec,
        out_shape=jax.ShapeDtypeStruct((T, H), jnp.float32),
    )(cnt, hidden_states, perm.reshape(E, R, RB), wsort.reshape(E, R, RB),
      Wg, bg.reshape(E, 1, FF), Wu, bu.reshape(E, 1, FF), Wd,
      bd.reshape(E, 1, H))
